# baseline (speedup 1.0000x reference)
import functools

import jax
import jax.numpy as jnp
from jax import lax
from jax.experimental import pallas as pl
from jax.experimental.pallas import tpu as pltpu
from jax.experimental.pallas import tpu_sc as plsc

N_HEADS = 4
N_OUT = 4
KC = 128
NC = 2
NS = 16
NW = NC * NS


def _edge_pass_body(cpw, cols, t_hbm, y_hbm, src_hbm, dst_hbm, c_hbm,
                    z_hbm, ones_hbm, out_hbm, ttab, y_vA, y_vB, m_vA,
                    m_vB, isrcA, isrcB, idstA, idstB, idstSA, idstSB,
                    c_v, acc, semYA, semYB, semIA, semIB, semSA, semSB):
  cid = lax.axis_index("c")
  sid = lax.axis_index("s")
  w = sid * NC + cid

  pltpu.sync_copy(t_hbm, ttab)
  pltpu.sync_copy(c_hbm, c_v)
  pltpu.sync_copy(ones_hbm, m_vA)
  pltpu.sync_copy(ones_hbm, m_vB)

  @pl.when(sid == 0)
  def _():
    pltpu.sync_copy(z_hbm, acc)

  plsc.subcore_barrier()

  iota = lax.iota(jnp.int32, 16)
  fh = [jnp.full((16,), h, jnp.int32) for h in range(N_HEADS)]
  csp = [c_v[h, :] for h in range(N_HEADS)]

  def fetch_idx(isrc, idst, semI, k):
    eb = (w * cpw + k) * KC
    pltpu.async_copy(src_hbm.at[pl.ds(eb, KC)], isrc, semI)
    pltpu.async_copy(dst_hbm.at[pl.ds(eb, KC)], idst, semI)

  def wait_idx(isrc, idst, semI, k):
    eb = (w * cpw + k) * KC
    pltpu.make_async_copy(src_hbm.at[pl.ds(eb, KC)], isrc, semI).wait()
    pltpu.make_async_copy(dst_hbm.at[pl.ds(eb, KC)], idst, semI).wait()

  def compute_scatter(y_v, isrc, idst, idstS, m_v, semS, pre_wait):
    if pre_wait:
      pltpu.make_async_copy(m_v, acc.at[idstS], semS).wait()
    for j in range(KC // 16):
      idstS[pl.ds(j * 16, 16)] = idst[pl.ds(j * 16, 16)]
    for g in range(KC // 16):
      s16 = isrc[pl.ds(g * 16, 16)]
      d16 = idst[pl.ds(g * 16, 16)]
      rows = iota + (g * 16)
      ts = [plsc.load_gather(ttab, [s16, fh[h]]) for h in range(N_HEADS)]
      td = [plsc.load_gather(ttab, [d16, fh[h]]) for h in range(N_HEADS)]
      d = [ts[h] - td[h] + csp[h] for h in range(N_HEADS)]
      mx = jnp.maximum(jnp.maximum(d[0], d[1]), jnp.maximum(d[2], d[3]))
      e = [jnp.exp(d[h] - mx) for h in range(N_HEADS)]
      ssum = (e[0] + e[1]) + (e[2] + e[3])
      q = [e[h] / ssum for h in range(N_HEADS)]
      for o in range(N_OUT):
        mo = q[0] * plsc.load_gather(
            y_v, [rows, jnp.full((16,), o, jnp.int32)])
        for h in range(1, N_HEADS):
          mo = mo + q[h] * plsc.load_gather(
              y_v, [rows, jnp.full((16,), h * N_OUT + o, jnp.int32)])
        plsc.store_scatter(m_v, [rows, jnp.full((16,), o, jnp.int32)], mo)
    pltpu.async_copy(m_v, acc.at[idstS], semS, add=True)

  bufs = ((y_vA, isrcA, idstA, idstSA, m_vA, semYA, semIA, semSA),
          (y_vB, isrcB, idstB, idstSB, m_vB, semYB, semIB, semSB))

  fetch_idx(isrcA, idstA, semIA, 0)
  wait_idx(isrcA, idstA, semIA, 0)
  pltpu.async_copy(y_hbm.at[isrcA], y_vA, semYA)
  fetch_idx(isrcB, idstB, semIB, 1)

  def pair(kk, pre_wait):
    for half in range(2):
      k = 2 * kk + half
      y_v, isrc, idst, idstS, m_v, semY, semI, semS = bufs[half]
      y_n, isrc_n, idst_n, _, _, semY_n, semI_n, _ = bufs[1 - half]
      wait_idx(isrc_n, idst_n, semI_n, jnp.minimum(k + 1, cpw - 1))
      pltpu.async_copy(y_hbm.at[isrc_n], y_n, semY_n)
      pltpu.make_async_copy(y_hbm.at[isrc], y_v, semY).wait()
      compute_scatter(y_v, isrc, idst, idstS, m_v, semS, pre_wait)
      fetch_idx(isrc, idst, semI, jnp.minimum(k + 2, cpw - 1))

  pair(0, False)
  lax.fori_loop(1, cpw // 2, lambda kk, c: (pair(kk, True), c)[1], 0)
  pltpu.make_async_copy(y_hbm.at[isrcA], y_vA, semYA).wait()
  wait_idx(isrcB, idstB, semIB, cpw - 1)
  pltpu.make_async_copy(m_vA, acc.at[idstSA], semSA).wait()
  pltpu.make_async_copy(m_vB, acc.at[idstSB], semSB).wait()

  plsc.subcore_barrier()

  @pl.when(sid == 0)
  def _():
    pltpu.sync_copy(acc, out_hbm.at[cid])


@functools.partial(jax.jit, static_argnames=("cpw", "first", "nt"))
def _edge_pass(t_pad, y_pad, src_p, dst_p, c_pad, cpw, first, nt):
  cols = 8
  mesh = plsc.VectorSubcoreMesh(
      core_axis_name="c", subcore_axis_name="s", num_cores=NC,
      num_subcores=NS)
  zeros = jnp.zeros((nt, cols), jnp.float32)
  ones = jnp.zeros((KC, cols), jnp.float32)
  if first:
    ones = ones.at[:, N_OUT:].set(1.0)
  body = functools.partial(_edge_pass_body, cpw, cols)
  return pl.kernel(
      body,
      out_type=jax.ShapeDtypeStruct((NC, nt, cols), jnp.float32),
      mesh=mesh,
      scratch_types=[
          pltpu.VMEM((nt, N_HEADS), jnp.float32),
          pltpu.VMEM((KC, N_HEADS * N_OUT), jnp.float32),
          pltpu.VMEM((KC, N_HEADS * N_OUT), jnp.float32),
          pltpu.VMEM((KC, cols), jnp.float32),
          pltpu.VMEM((KC, cols), jnp.float32),
          pltpu.VMEM((KC,), jnp.int32),
          pltpu.VMEM((KC,), jnp.int32),
          pltpu.VMEM((KC,), jnp.int32),
          pltpu.VMEM((KC,), jnp.int32),
          pltpu.VMEM((KC,), jnp.int32),
          pltpu.VMEM((KC,), jnp.int32),
          pltpu.VMEM((N_HEADS, 16), jnp.float32),
          pltpu.VMEM_SHARED((nt, cols), jnp.float32),
          pltpu.SemaphoreType.DMA,
          pltpu.SemaphoreType.DMA,
          pltpu.SemaphoreType.DMA,
          pltpu.SemaphoreType.DMA,
          pltpu.SemaphoreType.DMA,
          pltpu.SemaphoreType.DMA,
      ],
      compiler_params=pltpu.CompilerParams(
          needs_layout_passes=False, use_tc_tiling_on_sc=False),
  )(t_pad, y_pad, src_p, dst_p, c_pad, zeros, ones)


def _bn(x, gamma, beta, eps=1e-5):
  mu = x.mean(axis=0)
  var = x.var(axis=0)
  return (x - mu) / jnp.sqrt(var + eps) * gamma + beta


def kernel(x, edge_index, params):
  n, _ = x.shape
  e = edge_index.shape[1]
  src0, dst0 = edge_index[0], edge_index[1]
  mask = src0 != dst0
  loop = jnp.arange(n, dtype=src0.dtype)
  src = jnp.concatenate([src0, loop])
  dst = jnp.concatenate([jnp.where(mask, dst0, n), loop])

  ep_raw = e + n
  ep = ((ep_raw + 2 * NW * KC - 1) // (2 * NW * KC)) * (2 * NW * KC)
  nt = ((n + 1 + 15) // 16) * 16
  npad = ep - ep_raw
  pad_idx = n + (jnp.arange(npad, dtype=jnp.int32) % (nt - n))
  src_p = jnp.concatenate([src, pad_idx]).astype(jnp.int32)
  dst_p = jnp.concatenate([dst, pad_idx]).astype(jnp.int32)
  cpw = ep // (NW * KC)

  convs, bns, lins = params["convs"], params["bns"], params["lins"]

  cnt = None

  def feast(h, conv, first):
    nonlocal cnt
    W, u, c, b = conv
    t = h @ u
    y = h @ W
    t_pad = jnp.zeros((nt, N_HEADS), jnp.float32).at[:n].set(t)
    y_pad = jnp.zeros((nt, N_HEADS * N_OUT), jnp.float32).at[:n].set(y)
    c_pad = jnp.broadcast_to(c[:, None], (N_HEADS, 16)).astype(jnp.float32)
    out2 = _edge_pass(t_pad, y_pad, src_p, dst_p, c_pad,
                      cpw=cpw, first=first, nt=nt)
    ssum = out2[0] + out2[1]
    if first:
      cnt = jnp.maximum(ssum[:n, N_OUT], 1.0)
    s = ssum[:n, :N_OUT] / cnt[:, None] + b
    return jax.nn.relu(s)

  h = feast(x, convs[0], True)
  for i in range(1, 4):
    h = feast(h, convs[i], False)
  h = _bn(h, *bns[0])
  for blk in range(1, 5):
    r = h
    for i in range(4):
      h = feast(h, convs[4 * blk + i], False)
    h = r + _bn(h, *bns[blk])
  z = h
  for i, (W, b) in enumerate(lins):
    z = z @ W.T + b
    if i < 3:
      z = jax.nn.relu(z)
  return jax.nn.sigmoid(z)


# speedup vs baseline: 1.0396x; 1.0396x over previous
import functools

import jax
import jax.numpy as jnp
from jax import lax
from jax.experimental import pallas as pl
from jax.experimental.pallas import tpu as pltpu
from jax.experimental.pallas import tpu_sc as plsc

N_HEADS = 4
N_OUT = 4
KC = 128
NC = 2
NS = 16
NW = NC * NS


def _edge_pass_body(cpw, cols, t_hbm, y_hbm, src_hbm, dst_hbm, c_hbm,
                    z_hbm, ones_hbm, out_hbm, ttab, y_vA, y_vB, m_v,
                    isrc0, isrc1, isrc2, isrc3, idst0, idst1, idst2,
                    idst3, c_v, acc, semYA, semYB, semI0, semI1, semI2,
                    semI3):
  cid = lax.axis_index("c")
  sid = lax.axis_index("s")
  w = sid * NC + cid

  pltpu.sync_copy(t_hbm, ttab)
  pltpu.sync_copy(c_hbm, c_v)
  pltpu.sync_copy(ones_hbm, m_v)

  @pl.when(sid == 0)
  def _():
    pltpu.sync_copy(z_hbm, acc)

  plsc.subcore_barrier()

  iota = lax.iota(jnp.int32, 16)
  fh = [jnp.full((16,), h, jnp.int32) for h in range(N_HEADS)]
  csp = [c_v[h, :] for h in range(N_HEADS)]

  def fetch_idx(isrc, idst, semI, k):
    eb = (w * cpw + k) * KC
    pltpu.async_copy(src_hbm.at[pl.ds(eb, KC)], isrc, semI)
    pltpu.async_copy(dst_hbm.at[pl.ds(eb, KC)], idst, semI)

  def wait_idx(isrc, idst, semI, k):
    eb = (w * cpw + k) * KC
    pltpu.make_async_copy(src_hbm.at[pl.ds(eb, KC)], isrc, semI).wait()
    pltpu.make_async_copy(dst_hbm.at[pl.ds(eb, KC)], idst, semI).wait()

  def compute_scatter(y_v, isrc, idst):
    for g in range(KC // 16):
      s16 = isrc[pl.ds(g * 16, 16)]
      d16 = idst[pl.ds(g * 16, 16)]
      rows = iota + (g * 16)
      ts = [plsc.load_gather(ttab, [s16, fh[h]]) for h in range(N_HEADS)]
      td = [plsc.load_gather(ttab, [d16, fh[h]]) for h in range(N_HEADS)]
      d = [ts[h] - td[h] + csp[h] for h in range(N_HEADS)]
      mx = jnp.maximum(jnp.maximum(d[0], d[1]), jnp.maximum(d[2], d[3]))
      e = [jnp.exp(d[h] - mx) for h in range(N_HEADS)]
      ssum = (e[0] + e[1]) + (e[2] + e[3])
      q = [e[h] / ssum for h in range(N_HEADS)]
      for o in range(N_OUT):
        mo = q[0] * plsc.load_gather(
            y_v, [rows, jnp.full((16,), o, jnp.int32)])
        for h in range(1, N_HEADS):
          mo = mo + q[h] * plsc.load_gather(
              y_v, [rows, jnp.full((16,), h * N_OUT + o, jnp.int32)])
        plsc.store_scatter(m_v, [rows, jnp.full((16,), o, jnp.int32)], mo)
    pltpu.sync_copy(m_v, acc.at[idst], add=True)

  idxsets = ((isrc0, idst0, semI0), (isrc1, idst1, semI1),
             (isrc2, idst2, semI2), (isrc3, idst3, semI3))
  ybufs = ((y_vA, semYA), (y_vB, semYB))

  fetch_idx(isrc0, idst0, semI0, 0)
  wait_idx(isrc0, idst0, semI0, 0)
  pltpu.async_copy(y_hbm.at[isrc0], y_vA, semYA)
  fetch_idx(isrc1, idst1, semI1, 1)
  fetch_idx(isrc2, idst2, semI2, 2)

  def step(jj, carry):
    for q in range(4):
      k = 4 * jj + q
      isrc, idst, semI = idxsets[q]
      isrc_n, idst_n, semI_n = idxsets[(q + 1) % 4]
      y_v, semY = ybufs[q % 2]
      y_n, semY_n = ybufs[(q + 1) % 2]
      wait_idx(isrc_n, idst_n, semI_n, jnp.minimum(k + 1, cpw - 1))
      pltpu.async_copy(y_hbm.at[isrc_n], y_n, semY_n)
      pltpu.make_async_copy(y_hbm.at[isrc], y_v, semY).wait()
      compute_scatter(y_v, isrc, idst)
      i3, d3, s3 = idxsets[(q + 3) % 4]
      fetch_idx(i3, d3, s3, jnp.minimum(k + 3, cpw - 1))
    return carry

  lax.fori_loop(0, cpw // 4, step, 0)
  pltpu.make_async_copy(y_hbm.at[isrc0], y_vA, semYA).wait()
  wait_idx(isrc1, idst1, semI1, cpw - 1)
  wait_idx(isrc2, idst2, semI2, cpw - 1)

  plsc.subcore_barrier()

  @pl.when(sid == 0)
  def _():
    pltpu.sync_copy(acc, out_hbm.at[cid])


@functools.partial(jax.jit, static_argnames=("cpw", "first", "nt"))
def _edge_pass(t_pad, y_pad, src_p, dst_p, c_pad, cpw, first, nt):
  cols = 8
  mesh = plsc.VectorSubcoreMesh(
      core_axis_name="c", subcore_axis_name="s", num_cores=NC,
      num_subcores=NS)
  zeros = jnp.zeros((nt, cols), jnp.float32)
  ones = jnp.zeros((KC, cols), jnp.float32)
  if first:
    ones = ones.at[:, N_OUT:].set(1.0)
  body = functools.partial(_edge_pass_body, cpw, cols)
  return pl.kernel(
      body,
      out_type=jax.ShapeDtypeStruct((NC, nt, cols), jnp.float32),
      mesh=mesh,
      scratch_types=[
          pltpu.VMEM((nt, N_HEADS), jnp.float32),
          pltpu.VMEM((KC, N_HEADS * N_OUT), jnp.float32),
          pltpu.VMEM((KC, N_HEADS * N_OUT), jnp.float32),
          pltpu.VMEM((KC, cols), jnp.float32),
          pltpu.VMEM((KC,), jnp.int32),
          pltpu.VMEM((KC,), jnp.int32),
          pltpu.VMEM((KC,), jnp.int32),
          pltpu.VMEM((KC,), jnp.int32),
          pltpu.VMEM((KC,), jnp.int32),
          pltpu.VMEM((KC,), jnp.int32),
          pltpu.VMEM((KC,), jnp.int32),
          pltpu.VMEM((KC,), jnp.int32),
          pltpu.VMEM((N_HEADS, 16), jnp.float32),
          pltpu.VMEM_SHARED((nt, cols), jnp.float32),
          pltpu.SemaphoreType.DMA,
          pltpu.SemaphoreType.DMA,
          pltpu.SemaphoreType.DMA,
          pltpu.SemaphoreType.DMA,
          pltpu.SemaphoreType.DMA,
          pltpu.SemaphoreType.DMA,
      ],
      compiler_params=pltpu.CompilerParams(
          needs_layout_passes=False, use_tc_tiling_on_sc=False),
  )(t_pad, y_pad, src_p, dst_p, c_pad, zeros, ones)


def _bn(x, gamma, beta, eps=1e-5):
  mu = x.mean(axis=0)
  var = x.var(axis=0)
  return (x - mu) / jnp.sqrt(var + eps) * gamma + beta


def kernel(x, edge_index, params):
  n, _ = x.shape
  e = edge_index.shape[1]
  src0, dst0 = edge_index[0], edge_index[1]
  mask = src0 != dst0
  loop = jnp.arange(n, dtype=src0.dtype)
  src = jnp.concatenate([src0, loop])
  dst = jnp.concatenate([jnp.where(mask, dst0, n), loop])

  ep_raw = e + n
  ep = ((ep_raw + 4 * NW * KC - 1) // (4 * NW * KC)) * (4 * NW * KC)
  nt = ((n + 1 + 15) // 16) * 16
  npad = ep - ep_raw
  pad_idx = n + (jnp.arange(npad, dtype=jnp.int32) % (nt - n))
  src_p = jnp.concatenate([src, pad_idx]).astype(jnp.int32)
  dst_p = jnp.concatenate([dst, pad_idx]).astype(jnp.int32)
  cpw = ep // (NW * KC)

  convs, bns, lins = params["convs"], params["bns"], params["lins"]

  cnt = None

  def feast(h, conv, first):
    nonlocal cnt
    W, u, c, b = conv
    t = h @ u
    y = h @ W
    t_pad = jnp.zeros((nt, N_HEADS), jnp.float32).at[:n].set(t)
    y_pad = jnp.zeros((nt, N_HEADS * N_OUT), jnp.float32).at[:n].set(y)
    c_pad = jnp.broadcast_to(c[:, None], (N_HEADS, 16)).astype(jnp.float32)
    out2 = _edge_pass(t_pad, y_pad, src_p, dst_p, c_pad,
                      cpw=cpw, first=first, nt=nt)
    ssum = out2[0] + out2[1]
    if first:
      cnt = jnp.maximum(ssum[:n, N_OUT], 1.0)
    s = ssum[:n, :N_OUT] / cnt[:, None] + b
    return jax.nn.relu(s)

  h = feast(x, convs[0], True)
  for i in range(1, 4):
    h = feast(h, convs[i], False)
  h = _bn(h, *bns[0])
  for blk in range(1, 5):
    r = h
    for i in range(4):
      h = feast(h, convs[4 * blk + i], False)
    h = r + _bn(h, *bns[blk])
  z = h
  for i, (W, b) in enumerate(lins):
    z = z @ W.T + b
    if i < 3:
      z = jax.nn.relu(z)
  return jax.nn.sigmoid(z)


# speedup vs baseline: 1.0767x; 1.0357x over previous
import functools

import jax
import jax.numpy as jnp
from jax import lax
from jax.experimental import pallas as pl
from jax.experimental.pallas import tpu as pltpu
from jax.experimental.pallas import tpu_sc as plsc

N_HEADS = 4
N_OUT = 4
KC = 128
NC = 2
NS = 16
NW = NC * NS


def _edge_pass_body(cpw, cols, t_hbm, y_hbm, src_hbm, dst_hbm, c_hbm,
                    z_hbm, ones_hbm, out_hbm, ttab, y_vA, y_vB, m_v,
                    isrcA, isrcB, idstA, idstB, c_v, acc,
                    semYA, semYB, semIA, semIB):
  cid = lax.axis_index("c")
  sid = lax.axis_index("s")
  w = sid * NC + cid

  pltpu.sync_copy(t_hbm, ttab)
  pltpu.sync_copy(c_hbm, c_v)
  pltpu.sync_copy(ones_hbm, m_v)

  @pl.when(sid == 0)
  def _():
    pltpu.sync_copy(z_hbm, acc)

  plsc.subcore_barrier()

  iota = lax.iota(jnp.int32, 16)
  fh = [jnp.full((16,), h, jnp.int32) for h in range(N_HEADS)]
  csp = [c_v[h, :] for h in range(N_HEADS)]

  def fetch_idx(isrc, idst, semI, k):
    eb = (w * cpw + k) * KC
    pltpu.async_copy(src_hbm.at[pl.ds(eb, KC)], isrc, semI)
    pltpu.async_copy(dst_hbm.at[pl.ds(eb, KC)], idst, semI)

  def wait_idx(isrc, idst, semI, k):
    eb = (w * cpw + k) * KC
    pltpu.make_async_copy(src_hbm.at[pl.ds(eb, KC)], isrc, semI).wait()
    pltpu.make_async_copy(dst_hbm.at[pl.ds(eb, KC)], idst, semI).wait()

  def compute_scatter(y_v, isrc, idst):
    for g in range(KC // 16):
      s16 = isrc[pl.ds(g * 16, 16)]
      d16 = idst[pl.ds(g * 16, 16)]
      rows = iota + (g * 16)
      ts = [plsc.load_gather(ttab, [s16, fh[h]]) for h in range(N_HEADS)]
      td = [plsc.load_gather(ttab, [d16, fh[h]]) for h in range(N_HEADS)]
      d = [ts[h] - td[h] + csp[h] for h in range(N_HEADS)]
      mx = jnp.maximum(jnp.maximum(d[0], d[1]), jnp.maximum(d[2], d[3]))
      e = [jnp.exp(d[h] - mx) for h in range(N_HEADS)]
      ssum = (e[0] + e[1]) + (e[2] + e[3])
      q = [e[h] / ssum for h in range(N_HEADS)]
      for o in range(N_OUT):
        mo = q[0] * plsc.load_gather(
            y_v, [rows, jnp.full((16,), o, jnp.int32)])
        for h in range(1, N_HEADS):
          mo = mo + q[h] * plsc.load_gather(
              y_v, [rows, jnp.full((16,), h * N_OUT + o, jnp.int32)])
        plsc.store_scatter(m_v, [rows, jnp.full((16,), o, jnp.int32)], mo)
    pltpu.sync_copy(m_v, acc.at[idst], add=True)

  bufs = ((y_vA, isrcA, idstA, semYA, semIA),
          (y_vB, isrcB, idstB, semYB, semIB))

  fetch_idx(isrcA, idstA, semIA, 0)
  wait_idx(isrcA, idstA, semIA, 0)
  pltpu.async_copy(y_hbm.at[isrcA], y_vA, semYA)
  fetch_idx(isrcB, idstB, semIB, 1)

  def step(kk, carry):
    for half in range(2):
      k = 2 * kk + half
      y_v, isrc, idst, semY, semI = bufs[half]
      y_n, isrc_n, idst_n, semY_n, semI_n = bufs[1 - half]
      wait_idx(isrc_n, idst_n, semI_n, jnp.minimum(k + 1, cpw - 1))
      pltpu.async_copy(y_hbm.at[isrc_n], y_n, semY_n)
      pltpu.make_async_copy(y_hbm.at[isrc], y_v, semY).wait()
      compute_scatter(y_v, isrc, idst)
      fetch_idx(isrc, idst, semI, jnp.minimum(k + 2, cpw - 1))
    return carry

  lax.fori_loop(0, cpw // 2, step, 0)
  pltpu.make_async_copy(y_hbm.at[isrcA], y_vA, semYA).wait()
  wait_idx(isrcB, idstB, semIB, cpw - 1)

  plsc.subcore_barrier()

  @pl.when(sid == 0)
  def _():
    pltpu.sync_copy(acc, out_hbm.at[cid])


@functools.partial(jax.jit, static_argnames=("cpw", "first", "nt"))
def _edge_pass(t_pad, y_pad, src_p, dst_p, c_pad, cpw, first, nt):
  cols = 8
  mesh = plsc.VectorSubcoreMesh(
      core_axis_name="c", subcore_axis_name="s", num_cores=NC,
      num_subcores=NS)
  zeros = jnp.zeros((nt, cols), jnp.float32)
  ones = jnp.zeros((KC, cols), jnp.float32)
  if first:
    ones = ones.at[:, N_OUT:].set(1.0)
  body = functools.partial(_edge_pass_body, cpw, cols)
  return pl.kernel(
      body,
      out_type=jax.ShapeDtypeStruct((NC, nt, cols), jnp.float32),
      mesh=mesh,
      scratch_types=[
          pltpu.VMEM((nt, N_HEADS), jnp.float32),
          pltpu.VMEM((KC, N_HEADS * N_OUT), jnp.float32),
          pltpu.VMEM((KC, N_HEADS * N_OUT), jnp.float32),
          pltpu.VMEM((KC, cols), jnp.float32),
          pltpu.VMEM((KC,), jnp.int32),
          pltpu.VMEM((KC,), jnp.int32),
          pltpu.VMEM((KC,), jnp.int32),
          pltpu.VMEM((KC,), jnp.int32),
          pltpu.VMEM((N_HEADS, 16), jnp.float32),
          pltpu.VMEM_SHARED((nt, cols), jnp.float32),
          pltpu.SemaphoreType.DMA,
          pltpu.SemaphoreType.DMA,
          pltpu.SemaphoreType.DMA,
          pltpu.SemaphoreType.DMA,
      ],
      compiler_params=pltpu.CompilerParams(
          needs_layout_passes=False, use_tc_tiling_on_sc=False),
  )(t_pad, y_pad, src_p, dst_p, c_pad, zeros, ones)


def _bn(x, gamma, beta, eps=1e-5):
  mu = x.mean(axis=0)
  var = x.var(axis=0)
  return (x - mu) / jnp.sqrt(var + eps) * gamma + beta


def kernel(x, edge_index, params):
  n, _ = x.shape
  e = edge_index.shape[1]
  src0, dst0 = edge_index[0], edge_index[1]
  mask = src0 != dst0
  loop = jnp.arange(n, dtype=src0.dtype)
  src = jnp.concatenate([src0, loop])
  dst = jnp.concatenate([jnp.where(mask, dst0, n), loop])

  ep_raw = e + n
  ep = ((ep_raw + 2 * NW * KC - 1) // (2 * NW * KC)) * (2 * NW * KC)
  nt = ((n + 1 + 15) // 16) * 16
  npad = ep - ep_raw
  pad_idx = n + (jnp.arange(npad, dtype=jnp.int32) % (nt - n))
  src_p = jnp.concatenate([src, pad_idx]).astype(jnp.int32)
  dst_p = jnp.concatenate([dst, pad_idx]).astype(jnp.int32)
  cpw = ep // (NW * KC)

  convs, bns, lins = params["convs"], params["bns"], params["lins"]

  cnt = None

  def feast(h, conv, first):
    nonlocal cnt
    W, u, c, b = conv
    t = h @ u
    y = h @ W
    t_pad = jnp.zeros((nt, N_HEADS), jnp.float32).at[:n].set(t)
    y_pad = jnp.zeros((nt, N_HEADS * N_OUT), jnp.float32).at[:n].set(y)
    c_pad = jnp.broadcast_to(c[:, None], (N_HEADS, 16)).astype(jnp.float32)
    out2 = _edge_pass(t_pad, y_pad, src_p, dst_p, c_pad,
                      cpw=cpw, first=first, nt=nt)
    ssum = out2[0] + out2[1]
    if first:
      cnt = jnp.maximum(ssum[:n, N_OUT], 1.0)
    s = ssum[:n, :N_OUT] / cnt[:, None] + b
    return jax.nn.relu(s)

  h = feast(x, convs[0], True)
  for i in range(1, 4):
    h = feast(h, convs[i], False)
  h = _bn(h, *bns[0])
  for blk in range(1, 5):
    r = h
    for i in range(4):
      h = feast(h, convs[4 * blk + i], False)
    h = r + _bn(h, *bns[blk])
  z = h
  for i, (W, b) in enumerate(lins):
    z = z @ W.T + b
    if i < 3:
      z = jax.nn.relu(z)
  return jax.nn.sigmoid(z)
